# baseline (speedup 1.0000x reference)
import numpy as np
import jax
import jax.numpy as jnp
from jax.experimental import pallas as pl
from jax.experimental.pallas import tpu as pltpu

_HIDDEN = 128
_FILT = 128
_NG = 50
_NI = 6
_CUT = 10.0
_MAXZ = 100
_N = 64
_LN2 = 0.6931471805599453

_OFFS = np.linspace(0.0, _CUT, _NG).astype(np.float32)
_COEFF = float(-0.5 / (_OFFS[1] - _OFFS[0]) ** 2)

_HI = jax.lax.Precision.HIGHEST


def _ssp(x):
    return jnp.maximum(x, 0.0) + jnp.log1p(jnp.exp(-jnp.abs(x))) - _LN2


def _pair_kernel(zc_ref, pos_ref, emb_ref, w1_ref, b1_ref, w2_ref, b2_ref,
                 l1_ref, l2_ref, bl2_ref, l_ref, bl_ref,
                 o1_ref, bo1_ref, o2_ref, bo2_ref,
                 h1w_ref, h1b_ref, h2w_ref, h2b_ref, out_ref):
    f32 = jnp.float32
    N = _N
    NN = N * N
    offs = (jax.lax.broadcasted_iota(jnp.int32, (1, _NG), 1).astype(f32)
            * np.float32(_CUT / (_NG - 1)))
    pos = pos_ref[0]

    same = (jax.lax.broadcasted_iota(jnp.int32, (N, N, 1), 0)
            == jax.lax.broadcasted_iota(jnp.int32, (N, N, 1), 1)).reshape(NN, 1)

    u_list = []
    c_list = []
    for c in range(2):
        p = pos[c]
        pi = jnp.broadcast_to(p.reshape(N, 1, 3), (N, N, 3)).reshape(NN, 3)
        pj = jnp.broadcast_to(p.reshape(1, N, 3), (N, N, 3)).reshape(NN, 3)
        diff = pi - pj
        d = jnp.sqrt(jnp.sum(diff * diff, axis=1, keepdims=True) + 1e-12)
        maskf = jnp.where((d < _CUT) & (~same), 1.0, 0.0).astype(f32)
        cc = 0.5 * (jnp.cos(d * (np.pi / _CUT)) + 1.0) * maskf
        u_list.append(_COEFF * (d - offs) ** 2)
        c_list.append(cc)
    rbf = jnp.exp(jnp.concatenate(u_list, axis=1)).astype(jnp.bfloat16)
    ccat = jnp.concatenate(
        [jnp.broadcast_to(c_list[0], (NN, _FILT)),
         jnp.broadcast_to(c_list[1], (NN, _FILT))], axis=1)

    zc = zc_ref[0]
    ioz = jax.lax.broadcasted_iota(jnp.int32, (N, _MAXZ), 1)
    ohc = jnp.concatenate(
        [(zc[0] == ioz).astype(f32), (zc[1] == ioz).astype(f32)], axis=1)
    h = jax.lax.dot_general(ohc, emb_ref[:, :], (((1,), (0,)), ((), ())),
                            preferred_element_type=f32, precision=_HI)

    bf16 = jnp.bfloat16
    for i in range(_NI):
        xj = jnp.dot(h, l1_ref[i], preferred_element_type=f32)
        w = _ssp(jnp.dot(rbf, w1_ref[i], preferred_element_type=f32) + b1_ref[i])
        w = jnp.dot(w.astype(bf16), w2_ref[i], preferred_element_type=f32) + b2_ref[i]
        w = w * ccat
        agg = jnp.sum(w.reshape(N, N, 2 * _FILT) * xj[None, :, :], axis=1)
        m = _ssp(jnp.dot(agg, l2_ref[i], preferred_element_type=f32) + bl2_ref[i])
        m = jnp.dot(m, l_ref[i], preferred_element_type=f32) + bl_ref[i]
        h = h + m

    o = _ssp(jnp.dot(h, o1_ref[:, :], preferred_element_type=f32) + bo1_ref[:, :])
    s = jnp.sum(o, axis=0, keepdims=True)
    e = (jnp.dot(s, o2_ref[:, :], preferred_element_type=f32, precision=_HI)
         + float(N) * bo2_ref[:, :])
    y = jnp.maximum(
        jnp.dot(e, h1w_ref[:, :], preferred_element_type=f32, precision=_HI)
        + h1b_ref[:, :], 0.0)
    y = (jnp.dot(y, h2w_ref[:, :], preferred_element_type=f32, precision=_HI)
         + h2b_ref[:, :])
    out_ref[:, :, :] = y.reshape(1, 1, 1)


def _bdiag(a, b):
    ka, na = a.shape
    kb, nb = b.shape
    return jnp.concatenate(
        [jnp.concatenate([a, jnp.zeros((ka, nb), jnp.float32)], 1),
         jnp.concatenate([jnp.zeros((kb, na), jnp.float32), b], 1)], 0)


def kernel(z, pos, params1, params2, head):
    B = z.shape[0]
    zq = z.reshape(B, 2, _N, 1).astype(jnp.int32)
    pq = pos.reshape(B, 2, _N, 3).astype(jnp.float32)

    i1 = params1["inter"]
    i2 = params2["inter"]
    W1s = jnp.stack([_bdiag(i1[i]["mlp1"]["w"], i2[i]["mlp1"]["w"]) for i in range(_NI)])
    B1s = jnp.stack([jnp.concatenate([i1[i]["mlp1"]["b"], i2[i]["mlp1"]["b"]])[None, :] for i in range(_NI)])
    W2s = jnp.stack([_bdiag(i1[i]["mlp2"]["w"], i2[i]["mlp2"]["w"]) for i in range(_NI)])
    B2s = jnp.stack([jnp.concatenate([i1[i]["mlp2"]["b"], i2[i]["mlp2"]["b"]])[None, :] for i in range(_NI)])
    L1s = jnp.stack([_bdiag(i1[i]["lin1"]["w"], i2[i]["lin1"]["w"]) for i in range(_NI)])
    L2s = jnp.stack([_bdiag(i1[i]["lin2"]["w"], i2[i]["lin2"]["w"]) for i in range(_NI)])
    BL2s = jnp.stack([jnp.concatenate([i1[i]["lin2"]["b"], i2[i]["lin2"]["b"]])[None, :] for i in range(_NI)])
    Ls = jnp.stack([_bdiag(i1[i]["lin"]["w"], i2[i]["lin"]["w"]) for i in range(_NI)])
    BLs = jnp.stack([jnp.concatenate([i1[i]["lin"]["b"], i2[i]["lin"]["b"]])[None, :] for i in range(_NI)])
    EMB = _bdiag(params1["embed"], params2["embed"])
    O1 = _bdiag(params1["out1"]["w"], params2["out1"]["w"])
    BO1 = jnp.concatenate([params1["out1"]["b"], params2["out1"]["b"]])[None, :]
    O2 = _bdiag(params1["out2"]["w"], params2["out2"]["w"])
    BO2 = jnp.concatenate([params1["out2"]["b"], params2["out2"]["b"]])[None, :]
    H1W = head["l1"]["w"]
    H1B = head["l1"]["b"][None, :]
    H2W = head["l2"]["w"]
    H2B = head["l2"]["b"][None, :]

    def full(a):
        return pl.BlockSpec(a.shape, lambda b, nd=a.ndim: (0,) * nd)

    bf16 = jnp.bfloat16
    W1s = W1s.astype(bf16)
    W2s = W2s.astype(bf16)
    consts = (EMB, W1s, B1s, W2s, B2s, L1s, L2s, BL2s, Ls, BLs,
              O1, BO1, O2, BO2, H1W, H1B, H2W, H2B)
    out = pl.pallas_call(
        _pair_kernel,
        grid=(B,),
        in_specs=[
            pl.BlockSpec((1, 2, _N, 1), lambda b: (b, 0, 0, 0)),
            pl.BlockSpec((1, 2, _N, 3), lambda b: (b, 0, 0, 0)),
        ] + [full(a) for a in consts],
        out_specs=pl.BlockSpec((1, 1, 1), lambda b: (b, 0, 0)),
        out_shape=jax.ShapeDtypeStruct((B, 1, 1), jnp.float32),
        compiler_params=pltpu.CompilerParams(dimension_semantics=("arbitrary",)),
    )(zq, pq, *consts)
    return out.reshape(B, 1)


# speedup vs baseline: 1.7868x; 1.7868x over previous
import numpy as np
import jax
import jax.numpy as jnp
from jax.experimental import pallas as pl
from jax.experimental.pallas import tpu as pltpu

_HIDDEN = 128
_FILT = 128
_NG = 50
_NI = 6
_CUT = 10.0
_MAXZ = 100
_N = 64
_LN2 = 0.6931471805599453

_OFFS = np.linspace(0.0, _CUT, _NG).astype(np.float32)
_COEFF = float(-0.5 / (_OFFS[1] - _OFFS[0]) ** 2)

_HI = jax.lax.Precision.HIGHEST

_COS_COEF = (0.99999999228596, -4.934801387623153, 4.058698250549149,
             -1.3351743915873315, 0.23506322961458181, -0.0253909641009894,
             0.001605306471105794)


def _cos_cut(d):
    s = d * d * (1.0 / (_CUT * _CUT))
    p = jnp.float32(_COS_COEF[6])
    for k in (5, 4, 3, 2, 1, 0):
        p = p * s + _COS_COEF[k]
    return 0.5 * (p + 1.0)


def _ssp(x):
    return jnp.maximum(x, 0.0) + jnp.log1p(jnp.exp(-jnp.abs(x))) - _LN2


def _pair_kernel(zc_ref, pos_ref, emb_ref, w1_ref, b1_ref, w2_ref, b2_ref,
                 l1_ref, l2_ref, bl2_ref, l_ref, bl_ref,
                 o1_ref, bo1_ref, o2_ref, bo2_ref,
                 h1w_ref, h1b_ref, h2w_ref, h2b_ref, out_ref):
    f32 = jnp.float32
    N = _N
    NN = N * N
    offs = (jax.lax.broadcasted_iota(jnp.int32, (1, _NG), 1).astype(f32)
            * np.float32(_CUT / (_NG - 1)))
    pos = pos_ref[0]

    same = (jax.lax.broadcasted_iota(jnp.int32, (N, N, 1), 0)
            == jax.lax.broadcasted_iota(jnp.int32, (N, N, 1), 1)).reshape(NN, 1)

    u_list = []
    c_list = []
    for c in range(2):
        p = pos[c]
        pi = jnp.broadcast_to(p.reshape(N, 1, 3), (N, N, 3)).reshape(NN, 3)
        pj = jnp.broadcast_to(p.reshape(1, N, 3), (N, N, 3)).reshape(NN, 3)
        diff = pi - pj
        d = jnp.sqrt(jnp.sum(diff * diff, axis=1, keepdims=True) + 1e-12)
        maskf = jnp.where((d < _CUT) & (~same), 1.0, 0.0).astype(f32)
        cc = _cos_cut(d) * maskf
        u_list.append(_COEFF * (d - offs) ** 2)
        c_list.append(cc)
    rbf = jnp.exp(jnp.concatenate(u_list, axis=1)).astype(jnp.bfloat16)
    ccat = jnp.concatenate(
        [jnp.broadcast_to(c_list[0], (NN, _FILT)),
         jnp.broadcast_to(c_list[1], (NN, _FILT))], axis=1)

    zc = zc_ref[0]
    ioz = jax.lax.broadcasted_iota(jnp.int32, (N, _MAXZ), 1)
    ohc = jnp.concatenate(
        [(zc[0] == ioz).astype(f32), (zc[1] == ioz).astype(f32)], axis=1)
    h = jax.lax.dot_general(ohc, emb_ref[:, :], (((1,), (0,)), ((), ())),
                            preferred_element_type=f32, precision=_HI)

    bf16 = jnp.bfloat16
    for i in range(_NI):
        xj = jnp.dot(h, l1_ref[i], preferred_element_type=f32)
        w = _ssp(jnp.dot(rbf, w1_ref[i], preferred_element_type=f32) + b1_ref[i])
        w = jnp.dot(w.astype(bf16), w2_ref[i], preferred_element_type=f32) + b2_ref[i]
        w = w * ccat
        agg = jnp.sum(w.reshape(N, N, 2 * _FILT) * xj[None, :, :], axis=1)
        m = _ssp(jnp.dot(agg, l2_ref[i], preferred_element_type=f32) + bl2_ref[i])
        m = jnp.dot(m, l_ref[i], preferred_element_type=f32) + bl_ref[i]
        h = h + m

    o = _ssp(jnp.dot(h, o1_ref[:, :], preferred_element_type=f32) + bo1_ref[:, :])
    s = jnp.sum(o, axis=0, keepdims=True)
    e = (jnp.dot(s, o2_ref[:, :], preferred_element_type=f32, precision=_HI)
         + float(N) * bo2_ref[:, :])
    y = jnp.maximum(
        jnp.dot(e, h1w_ref[:, :], preferred_element_type=f32, precision=_HI)
        + h1b_ref[:, :], 0.0)
    y = (jnp.dot(y, h2w_ref[:, :], preferred_element_type=f32, precision=_HI)
         + h2b_ref[:, :])
    out_ref[:, :, :] = y.reshape(1, 1, 1)


def _bdiag(a, b):
    ka, na = a.shape
    kb, nb = b.shape
    return jnp.concatenate(
        [jnp.concatenate([a, jnp.zeros((ka, nb), jnp.float32)], 1),
         jnp.concatenate([jnp.zeros((kb, na), jnp.float32), b], 1)], 0)


def kernel(z, pos, params1, params2, head):
    B = z.shape[0]
    zq = z.reshape(B, 2, _N, 1).astype(jnp.int32)
    pq = pos.reshape(B, 2, _N, 3).astype(jnp.float32)

    i1 = params1["inter"]
    i2 = params2["inter"]
    W1s = jnp.stack([_bdiag(i1[i]["mlp1"]["w"], i2[i]["mlp1"]["w"]) for i in range(_NI)])
    B1s = jnp.stack([jnp.concatenate([i1[i]["mlp1"]["b"], i2[i]["mlp1"]["b"]])[None, :] for i in range(_NI)])
    W2s = jnp.stack([_bdiag(i1[i]["mlp2"]["w"], i2[i]["mlp2"]["w"]) for i in range(_NI)])
    B2s = jnp.stack([jnp.concatenate([i1[i]["mlp2"]["b"], i2[i]["mlp2"]["b"]])[None, :] for i in range(_NI)])
    L1s = jnp.stack([_bdiag(i1[i]["lin1"]["w"], i2[i]["lin1"]["w"]) for i in range(_NI)])
    L2s = jnp.stack([_bdiag(i1[i]["lin2"]["w"], i2[i]["lin2"]["w"]) for i in range(_NI)])
    BL2s = jnp.stack([jnp.concatenate([i1[i]["lin2"]["b"], i2[i]["lin2"]["b"]])[None, :] for i in range(_NI)])
    Ls = jnp.stack([_bdiag(i1[i]["lin"]["w"], i2[i]["lin"]["w"]) for i in range(_NI)])
    BLs = jnp.stack([jnp.concatenate([i1[i]["lin"]["b"], i2[i]["lin"]["b"]])[None, :] for i in range(_NI)])
    EMB = _bdiag(params1["embed"], params2["embed"])
    O1 = _bdiag(params1["out1"]["w"], params2["out1"]["w"])
    BO1 = jnp.concatenate([params1["out1"]["b"], params2["out1"]["b"]])[None, :]
    O2 = _bdiag(params1["out2"]["w"], params2["out2"]["w"])
    BO2 = jnp.concatenate([params1["out2"]["b"], params2["out2"]["b"]])[None, :]
    H1W = head["l1"]["w"]
    H1B = head["l1"]["b"][None, :]
    H2W = head["l2"]["w"]
    H2B = head["l2"]["b"][None, :]

    def full(a):
        return pl.BlockSpec(a.shape, lambda b, nd=a.ndim: (0,) * nd)

    bf16 = jnp.bfloat16
    W1s = W1s.astype(bf16)
    W2s = W2s.astype(bf16)
    consts = (EMB, W1s, B1s, W2s, B2s, L1s, L2s, BL2s, Ls, BLs,
              O1, BO1, O2, BO2, H1W, H1B, H2W, H2B)
    out = pl.pallas_call(
        _pair_kernel,
        grid=(B,),
        in_specs=[
            pl.BlockSpec((1, 2, _N, 1), lambda b: (b, 0, 0, 0)),
            pl.BlockSpec((1, 2, _N, 3), lambda b: (b, 0, 0, 0)),
        ] + [full(a) for a in consts],
        out_specs=pl.BlockSpec((1, 1, 1), lambda b: (b, 0, 0)),
        out_shape=jax.ShapeDtypeStruct((B, 1, 1), jnp.float32),
        compiler_params=pltpu.CompilerParams(dimension_semantics=("arbitrary",)),
    )(zq, pq, *consts)
    return out.reshape(B, 1)


# speedup vs baseline: 2.8464x; 1.5930x over previous
import numpy as np
import jax
import jax.numpy as jnp
from jax.experimental import pallas as pl
from jax.experimental.pallas import tpu as pltpu

_HIDDEN = 128
_FILT = 128
_NG = 50
_NI = 6
_CUT = 10.0
_MAXZ = 100
_N = 64
_LN2 = 0.6931471805599453

_OFFS = np.linspace(0.0, _CUT, _NG).astype(np.float32)
_COEFF = float(-0.5 / (_OFFS[1] - _OFFS[0]) ** 2)

_TROWS = 128
_DELTA = float(_CUT / (_TROWS - 3))

_HI = jax.lax.Precision.HIGHEST

_COS_COEF = (0.99999999228596, -4.934801387623153, 4.058698250549149,
             -1.3351743915873315, 0.23506322961458181, -0.0253909641009894,
             0.001605306471105794)


def _cos_cut(d):
    s = d * d * (1.0 / (_CUT * _CUT))
    p = jnp.float32(_COS_COEF[6])
    for k in (5, 4, 3, 2, 1, 0):
        p = p * s + _COS_COEF[k]
    return 0.5 * (p + 1.0)


def _ssp(x):
    return jnp.maximum(x, 0.0) + jnp.log1p(jnp.exp(-jnp.abs(x))) - _LN2


def _table_kernel(w1_ref, b1_ref, w2_ref, b2_ref, t_ref):
    f32 = jnp.float32
    offs = (jax.lax.broadcasted_iota(jnp.int32, (1, _NG), 1).astype(f32)
            * np.float32(_CUT / (_NG - 1)))
    dk = (jax.lax.broadcasted_iota(jnp.int32, (_TROWS, 1), 0).astype(f32)
          - 1.0) * np.float32(_DELTA)
    rb = jnp.exp(_COEFF * (dk - offs) ** 2)
    rbc = jnp.concatenate([rb, rb], axis=1)
    for i in range(_NI):
        t = _ssp(jnp.dot(rbc, w1_ref[i], preferred_element_type=f32,
                         precision=_HI) + b1_ref[i])
        t = jnp.dot(t, w2_ref[i], preferred_element_type=f32,
                    precision=_HI) + b2_ref[i]
        t_ref[i] = t


def _catmull_basis(t, riota):
    x = t - riota
    a = jnp.abs(x)
    a2 = a * a
    inner = (1.5 * a - 2.5) * a2 + 1.0
    outer = ((-0.5 * a + 2.5) * a - 4.0) * a + 2.0
    w = jnp.where(a < 1.0, inner, outer)
    return jnp.where(a < 2.0, w, 0.0)


def _pair_kernel(zc_ref, pos_ref, emb_ref, td_ref,
                 l1_ref, l2_ref, bl2_ref, l_ref, bl_ref,
                 o1_ref, bo1_ref, o2_ref, bo2_ref,
                 h1w_ref, h1b_ref, h2w_ref, h2b_ref, out_ref):
    f32 = jnp.float32
    bf16 = jnp.bfloat16
    N = _N
    NN = N * N
    pos = pos_ref[0]

    same = (jax.lax.broadcasted_iota(jnp.int32, (N, N, 1), 0)
            == jax.lax.broadcasted_iota(jnp.int32, (N, N, 1), 1)).reshape(NN, 1)
    riota = (jax.lax.broadcasted_iota(jnp.int32, (1, _TROWS), 1).astype(f32)
             - 1.0)

    b_list = []
    c_list = []
    for c in range(2):
        p = pos[c]
        pi = jnp.broadcast_to(p.reshape(N, 1, 3), (N, N, 3)).reshape(NN, 3)
        pj = jnp.broadcast_to(p.reshape(1, N, 3), (N, N, 3)).reshape(NN, 3)
        diff = pi - pj
        d = jnp.sqrt(jnp.sum(diff * diff, axis=1, keepdims=True) + 1e-12)
        maskf = jnp.where((d < _CUT) & (~same), 1.0, 0.0).astype(f32)
        cc = _cos_cut(d) * maskf
        b_list.append(_catmull_basis(d * np.float32(1.0 / _DELTA), riota))
        c_list.append(cc)
    bcat = jnp.concatenate(b_list, axis=1).astype(bf16)
    ccat = jnp.concatenate(
        [jnp.broadcast_to(c_list[0], (NN, _FILT)),
         jnp.broadcast_to(c_list[1], (NN, _FILT))], axis=1)

    zc = zc_ref[0]
    ioz = jax.lax.broadcasted_iota(jnp.int32, (N, _MAXZ), 1)
    ohc = jnp.concatenate(
        [(zc[0] == ioz).astype(f32), (zc[1] == ioz).astype(f32)], axis=1)
    h = jax.lax.dot_general(ohc, emb_ref[:, :], (((1,), (0,)), ((), ())),
                            preferred_element_type=f32, precision=_HI)

    for i in range(_NI):
        xj = jnp.dot(h, l1_ref[i], preferred_element_type=f32)
        w = jnp.dot(bcat, td_ref[i], preferred_element_type=f32)
        w = w * ccat
        agg = jnp.sum(w.reshape(N, N, 2 * _FILT) * xj[None, :, :], axis=1)
        m = _ssp(jnp.dot(agg, l2_ref[i], preferred_element_type=f32) + bl2_ref[i])
        m = jnp.dot(m, l_ref[i], preferred_element_type=f32) + bl_ref[i]
        h = h + m

    o = _ssp(jnp.dot(h, o1_ref[:, :], preferred_element_type=f32) + bo1_ref[:, :])
    s = jnp.sum(o, axis=0, keepdims=True)
    e = (jnp.dot(s, o2_ref[:, :], preferred_element_type=f32, precision=_HI)
         + float(N) * bo2_ref[:, :])
    y = jnp.maximum(
        jnp.dot(e, h1w_ref[:, :], preferred_element_type=f32, precision=_HI)
        + h1b_ref[:, :], 0.0)
    y = (jnp.dot(y, h2w_ref[:, :], preferred_element_type=f32, precision=_HI)
         + h2b_ref[:, :])
    out_ref[:, :, :] = y.reshape(1, 1, 1)


def _bdiag(a, b):
    ka, na = a.shape
    kb, nb = b.shape
    return jnp.concatenate(
        [jnp.concatenate([a, jnp.zeros((ka, nb), jnp.float32)], 1),
         jnp.concatenate([jnp.zeros((kb, na), jnp.float32), b], 1)], 0)


def kernel(z, pos, params1, params2, head):
    B = z.shape[0]
    zq = z.reshape(B, 2, _N, 1).astype(jnp.int32)
    pq = pos.reshape(B, 2, _N, 3).astype(jnp.float32)

    i1 = params1["inter"]
    i2 = params2["inter"]
    W1s = jnp.stack([_bdiag(i1[i]["mlp1"]["w"], i2[i]["mlp1"]["w"]) for i in range(_NI)])
    B1s = jnp.stack([jnp.concatenate([i1[i]["mlp1"]["b"], i2[i]["mlp1"]["b"]])[None, :] for i in range(_NI)])
    W2s = jnp.stack([_bdiag(i1[i]["mlp2"]["w"], i2[i]["mlp2"]["w"]) for i in range(_NI)])
    B2s = jnp.stack([jnp.concatenate([i1[i]["mlp2"]["b"], i2[i]["mlp2"]["b"]])[None, :] for i in range(_NI)])
    L1s = jnp.stack([_bdiag(i1[i]["lin1"]["w"], i2[i]["lin1"]["w"]) for i in range(_NI)])
    L2s = jnp.stack([_bdiag(i1[i]["lin2"]["w"], i2[i]["lin2"]["w"]) for i in range(_NI)])
    BL2s = jnp.stack([jnp.concatenate([i1[i]["lin2"]["b"], i2[i]["lin2"]["b"]])[None, :] for i in range(_NI)])
    Ls = jnp.stack([_bdiag(i1[i]["lin"]["w"], i2[i]["lin"]["w"]) for i in range(_NI)])
    BLs = jnp.stack([jnp.concatenate([i1[i]["lin"]["b"], i2[i]["lin"]["b"]])[None, :] for i in range(_NI)])
    EMB = _bdiag(params1["embed"], params2["embed"])
    O1 = _bdiag(params1["out1"]["w"], params2["out1"]["w"])
    BO1 = jnp.concatenate([params1["out1"]["b"], params2["out1"]["b"]])[None, :]
    O2 = _bdiag(params1["out2"]["w"], params2["out2"]["w"])
    BO2 = jnp.concatenate([params1["out2"]["b"], params2["out2"]["b"]])[None, :]
    H1W = head["l1"]["w"]
    H1B = head["l1"]["b"][None, :]
    H2W = head["l2"]["w"]
    H2B = head["l2"]["b"][None, :]

    def full(a):
        return pl.BlockSpec(a.shape, lambda b, nd=a.ndim: (0,) * nd)

    tconsts = (W1s, B1s, W2s, B2s)

    def full0(a):
        return pl.BlockSpec(a.shape, lambda nd=a.ndim: (0,) * nd)

    T = pl.pallas_call(
        _table_kernel,
        in_specs=[full0(a) for a in tconsts],
        out_specs=pl.BlockSpec((_NI, _TROWS, 2 * _FILT), lambda: (0, 0, 0)),
        out_shape=jax.ShapeDtypeStruct((_NI, _TROWS, 2 * _FILT), jnp.float32),
    )(*tconsts)

    bf16 = jnp.bfloat16
    zpad = jnp.zeros((_NI, _TROWS, _FILT), jnp.float32)
    TD = jnp.concatenate(
        [jnp.concatenate([T[:, :, :_FILT], zpad], axis=2),
         jnp.concatenate([zpad, T[:, :, _FILT:]], axis=2)], axis=1).astype(bf16)

    consts = (EMB, TD, L1s, L2s, BL2s, Ls, BLs,
              O1, BO1, O2, BO2, H1W, H1B, H2W, H2B)
    out = pl.pallas_call(
        _pair_kernel,
        grid=(B,),
        in_specs=[
            pl.BlockSpec((1, 2, _N, 1), lambda b: (b, 0, 0, 0)),
            pl.BlockSpec((1, 2, _N, 3), lambda b: (b, 0, 0, 0)),
        ] + [full(a) for a in consts],
        out_specs=pl.BlockSpec((1, 1, 1), lambda b: (b, 0, 0)),
        out_shape=jax.ShapeDtypeStruct((B, 1, 1), jnp.float32),
        compiler_params=pltpu.CompilerParams(dimension_semantics=("arbitrary",)),
    )(zq, pq, *consts)
    return out.reshape(B, 1)


# speedup vs baseline: 2.8671x; 1.0073x over previous
import numpy as np
import jax
import jax.numpy as jnp
from jax.experimental import pallas as pl
from jax.experimental.pallas import tpu as pltpu

_HIDDEN = 128
_FILT = 128
_NG = 50
_NI = 6
_CUT = 10.0
_MAXZ = 100
_N = 64
_LN2 = 0.6931471805599453

_OFFS = np.linspace(0.0, _CUT, _NG).astype(np.float32)
_COEFF = float(-0.5 / (_OFFS[1] - _OFFS[0]) ** 2)

_TROWS = 128
_DELTA = float(_CUT / (_TROWS - 3))

_HI = jax.lax.Precision.HIGHEST

_COS_COEF = (0.99999999228596, -4.934801387623153, 4.058698250549149,
             -1.3351743915873315, 0.23506322961458181, -0.0253909641009894,
             0.001605306471105794)


def _cos_cut(d):
    s = d * d * (1.0 / (_CUT * _CUT))
    p = jnp.float32(_COS_COEF[6])
    for k in (5, 4, 3, 2, 1, 0):
        p = p * s + _COS_COEF[k]
    return 0.5 * (p + 1.0)


def _ssp(x):
    return jnp.maximum(x, 0.0) + jnp.log1p(jnp.exp(-jnp.abs(x))) - _LN2


def _table_kernel(w1_ref, b1_ref, w2_ref, b2_ref, t_ref):
    f32 = jnp.float32
    offs = (jax.lax.broadcasted_iota(jnp.int32, (1, _NG), 1).astype(f32)
            * np.float32(_CUT / (_NG - 1)))
    dk = (jax.lax.broadcasted_iota(jnp.int32, (_TROWS, 1), 0).astype(f32)
          - 1.0) * np.float32(_DELTA)
    rb = jnp.exp(_COEFF * (dk - offs) ** 2)
    rbc = jnp.concatenate([rb, rb], axis=1)
    zpad = jnp.zeros((_TROWS, _FILT), jnp.bfloat16)
    for i in range(_NI):
        t = _ssp(jnp.dot(rbc, w1_ref[i], preferred_element_type=f32,
                         precision=_HI) + b1_ref[i])
        t = (jnp.dot(t, w2_ref[i], preferred_element_type=f32,
                     precision=_HI) + b2_ref[i]).astype(jnp.bfloat16)
        t_ref[i] = jnp.concatenate(
            [jnp.concatenate([t[:, :_FILT], zpad], axis=1),
             jnp.concatenate([zpad, t[:, _FILT:]], axis=1)], axis=0)


def _catmull_basis(t, riota):
    x = t - riota
    a = jnp.abs(x)
    a2 = a * a
    inner = (1.5 * a - 2.5) * a2 + 1.0
    outer = ((-0.5 * a + 2.5) * a - 4.0) * a + 2.0
    w = jnp.where(a < 1.0, inner, outer)
    return jnp.where(a < 2.0, w, 0.0)


def _pair_kernel(zc_ref, pos_ref, emb_ref, td_ref,
                 l1_ref, l2_ref, bl2_ref, l_ref, bl_ref,
                 o1_ref, bo1_ref, o2_ref, bo2_ref,
                 h1w_ref, h1b_ref, h2w_ref, h2b_ref, out_ref):
    f32 = jnp.float32
    bf16 = jnp.bfloat16
    N = _N
    NN = N * N
    pos = pos_ref[0]

    same = (jax.lax.broadcasted_iota(jnp.int32, (N, N, 1), 0)
            == jax.lax.broadcasted_iota(jnp.int32, (N, N, 1), 1)).reshape(NN, 1)
    riota = (jax.lax.broadcasted_iota(jnp.int32, (1, _TROWS), 1).astype(f32)
             - 1.0)

    b_list = []
    for c in range(2):
        p = pos[c]
        pi = jnp.broadcast_to(p.reshape(N, 1, 3), (N, N, 3)).reshape(NN, 3)
        pj = jnp.broadcast_to(p.reshape(1, N, 3), (N, N, 3)).reshape(NN, 3)
        diff = pi - pj
        d = jnp.sqrt(jnp.sum(diff * diff, axis=1, keepdims=True) + 1e-12)
        maskf = jnp.where((d < _CUT) & (~same), 1.0, 0.0).astype(f32)
        cc = _cos_cut(d) * maskf
        b_list.append(_catmull_basis(d * np.float32(1.0 / _DELTA), riota) * cc)
    bcat = jnp.concatenate(b_list, axis=1).astype(bf16)

    zc = zc_ref[0]
    ioz = jax.lax.broadcasted_iota(jnp.int32, (N, _MAXZ), 1)
    ohc = jnp.concatenate(
        [(zc[0] == ioz).astype(f32), (zc[1] == ioz).astype(f32)], axis=1)
    h = jax.lax.dot_general(ohc, emb_ref[:, :], (((1,), (0,)), ((), ())),
                            preferred_element_type=f32, precision=_HI)

    for i in range(_NI):
        xj = jnp.dot(h, l1_ref[i], preferred_element_type=f32)
        w = jnp.dot(bcat, td_ref[i], preferred_element_type=f32)
        agg = jnp.sum(w.reshape(N, N, 2 * _FILT) * xj[None, :, :], axis=1)
        m = _ssp(jnp.dot(agg, l2_ref[i], preferred_element_type=f32) + bl2_ref[i])
        m = jnp.dot(m, l_ref[i], preferred_element_type=f32) + bl_ref[i]
        h = h + m

    o = _ssp(jnp.dot(h, o1_ref[:, :], preferred_element_type=f32) + bo1_ref[:, :])
    s = jnp.sum(o, axis=0, keepdims=True)
    e = (jnp.dot(s, o2_ref[:, :], preferred_element_type=f32, precision=_HI)
         + float(N) * bo2_ref[:, :])
    y = jnp.maximum(
        jnp.dot(e, h1w_ref[:, :], preferred_element_type=f32, precision=_HI)
        + h1b_ref[:, :], 0.0)
    y = (jnp.dot(y, h2w_ref[:, :], preferred_element_type=f32, precision=_HI)
         + h2b_ref[:, :])
    out_ref[:, :, :] = y.reshape(1, 1, 1)


def _bdiag(a, b):
    ka, na = a.shape
    kb, nb = b.shape
    return jnp.concatenate(
        [jnp.concatenate([a, jnp.zeros((ka, nb), jnp.float32)], 1),
         jnp.concatenate([jnp.zeros((kb, na), jnp.float32), b], 1)], 0)


def kernel(z, pos, params1, params2, head):
    B = z.shape[0]
    zq = z.reshape(B, 2, _N, 1).astype(jnp.int32)
    pq = pos.reshape(B, 2, _N, 3).astype(jnp.float32)

    i1 = params1["inter"]
    i2 = params2["inter"]
    W1s = jnp.stack([_bdiag(i1[i]["mlp1"]["w"], i2[i]["mlp1"]["w"]) for i in range(_NI)])
    B1s = jnp.stack([jnp.concatenate([i1[i]["mlp1"]["b"], i2[i]["mlp1"]["b"]])[None, :] for i in range(_NI)])
    W2s = jnp.stack([_bdiag(i1[i]["mlp2"]["w"], i2[i]["mlp2"]["w"]) for i in range(_NI)])
    B2s = jnp.stack([jnp.concatenate([i1[i]["mlp2"]["b"], i2[i]["mlp2"]["b"]])[None, :] for i in range(_NI)])
    L1s = jnp.stack([_bdiag(i1[i]["lin1"]["w"], i2[i]["lin1"]["w"]) for i in range(_NI)])
    L2s = jnp.stack([_bdiag(i1[i]["lin2"]["w"], i2[i]["lin2"]["w"]) for i in range(_NI)])
    BL2s = jnp.stack([jnp.concatenate([i1[i]["lin2"]["b"], i2[i]["lin2"]["b"]])[None, :] for i in range(_NI)])
    Ls = jnp.stack([_bdiag(i1[i]["lin"]["w"], i2[i]["lin"]["w"]) for i in range(_NI)])
    BLs = jnp.stack([jnp.concatenate([i1[i]["lin"]["b"], i2[i]["lin"]["b"]])[None, :] for i in range(_NI)])
    EMB = _bdiag(params1["embed"], params2["embed"])
    O1 = _bdiag(params1["out1"]["w"], params2["out1"]["w"])
    BO1 = jnp.concatenate([params1["out1"]["b"], params2["out1"]["b"]])[None, :]
    O2 = _bdiag(params1["out2"]["w"], params2["out2"]["w"])
    BO2 = jnp.concatenate([params1["out2"]["b"], params2["out2"]["b"]])[None, :]
    H1W = head["l1"]["w"]
    H1B = head["l1"]["b"][None, :]
    H2W = head["l2"]["w"]
    H2B = head["l2"]["b"][None, :]

    def full(a):
        return pl.BlockSpec(a.shape, lambda b, nd=a.ndim: (0,) * nd)

    tconsts = (W1s, B1s, W2s, B2s)

    def full0(a):
        return pl.BlockSpec(a.shape, lambda nd=a.ndim: (0,) * nd)

    TD = pl.pallas_call(
        _table_kernel,
        in_specs=[full0(a) for a in tconsts],
        out_specs=pl.BlockSpec((_NI, 2 * _TROWS, 2 * _FILT), lambda: (0, 0, 0)),
        out_shape=jax.ShapeDtypeStruct((_NI, 2 * _TROWS, 2 * _FILT),
                                       jnp.bfloat16),
    )(*tconsts)

    consts = (EMB, TD, L1s, L2s, BL2s, Ls, BLs,
              O1, BO1, O2, BO2, H1W, H1B, H2W, H2B)
    out = pl.pallas_call(
        _pair_kernel,
        grid=(B,),
        in_specs=[
            pl.BlockSpec((1, 2, _N, 1), lambda b: (b, 0, 0, 0)),
            pl.BlockSpec((1, 2, _N, 3), lambda b: (b, 0, 0, 0)),
        ] + [full(a) for a in consts],
        out_specs=pl.BlockSpec((1, 1, 1), lambda b: (b, 0, 0)),
        out_shape=jax.ShapeDtypeStruct((B, 1, 1), jnp.float32),
        compiler_params=pltpu.CompilerParams(dimension_semantics=("parallel",)),
    )(zq, pq, *consts)
    return out.reshape(B, 1)


# speedup vs baseline: 3.1054x; 1.0831x over previous
import numpy as np
import jax
import jax.numpy as jnp
from jax.experimental import pallas as pl
from jax.experimental.pallas import tpu as pltpu

_HIDDEN = 128
_FILT = 128
_NG = 50
_NI = 6
_CUT = 10.0
_MAXZ = 100
_N = 64
_LN2 = 0.6931471805599453

_OFFS = np.linspace(0.0, _CUT, _NG).astype(np.float32)
_COEFF = float(-0.5 / (_OFFS[1] - _OFFS[0]) ** 2)

_TROWS = 128
_DELTA = float(_CUT / (_TROWS - 3))

_HI = jax.lax.Precision.HIGHEST

_COS_COEF = (0.99999999228596, -4.934801387623153, 4.058698250549149,
             -1.3351743915873315, 0.23506322961458181, -0.0253909641009894,
             0.001605306471105794)


def _cos_cut(d):
    s = d * d * (1.0 / (_CUT * _CUT))
    p = jnp.float32(_COS_COEF[6])
    for k in (5, 4, 3, 2, 1, 0):
        p = p * s + _COS_COEF[k]
    return 0.5 * (p + 1.0)


def _ssp(x):
    return jnp.maximum(x, 0.0) + jnp.log1p(jnp.exp(-jnp.abs(x))) - _LN2


def _table_kernel(w1_ref, b1_ref, w2_ref, b2_ref, t_ref):
    f32 = jnp.float32
    offs = (jax.lax.broadcasted_iota(jnp.int32, (1, _NG), 1).astype(f32)
            * np.float32(_CUT / (_NG - 1)))
    dk = (jax.lax.broadcasted_iota(jnp.int32, (_TROWS, 1), 0).astype(f32)
          - 1.0) * np.float32(_DELTA)
    rb = jnp.exp(_COEFF * (dk - offs) ** 2)
    rbc = jnp.concatenate([rb, rb], axis=1)
    zpad = jnp.zeros((_TROWS, _FILT), jnp.bfloat16)
    for i in range(_NI):
        t = _ssp(jnp.dot(rbc, w1_ref[i], preferred_element_type=f32,
                         precision=_HI) + b1_ref[i])
        t = (jnp.dot(t, w2_ref[i], preferred_element_type=f32,
                     precision=_HI) + b2_ref[i]).astype(jnp.bfloat16)
        t_ref[i] = jnp.concatenate(
            [jnp.concatenate([t[:, :_FILT], zpad], axis=1),
             jnp.concatenate([zpad, t[:, _FILT:]], axis=1)], axis=0)


def _catmull_basis(t, riota):
    x = t - riota
    a = jnp.abs(x)
    a2 = a * a
    inner = (1.5 * a - 2.5) * a2 + 1.0
    outer = ((-0.5 * a + 2.5) * a - 4.0) * a + 2.0
    w = jnp.where(a < 1.0, inner, outer)
    return jnp.where(a < 2.0, w, 0.0)


def _pair_kernel(zc_ref, pos_ref, emb_ref, td_ref,
                 l1_ref, l2_ref, bl2_ref, l_ref, bl_ref,
                 o1_ref, bo1_ref, o2_ref, bo2_ref,
                 h1w_ref, h1b_ref, h2w_ref, h2b_ref, out_ref):
    f32 = jnp.float32
    bf16 = jnp.bfloat16
    N = _N
    NN = N * N
    pos = pos_ref[0]

    same = (jax.lax.broadcasted_iota(jnp.int32, (N, N, 1), 0)
            == jax.lax.broadcasted_iota(jnp.int32, (N, N, 1), 1)).reshape(NN, 1)
    riota = (jax.lax.broadcasted_iota(jnp.int32, (1, _TROWS), 1).astype(f32)
             - 1.0)

    b_list = []
    for c in range(2):
        p = pos[c]
        pi = jnp.broadcast_to(p.reshape(N, 1, 3), (N, N, 3)).reshape(NN, 3)
        pj = jnp.broadcast_to(p.reshape(1, N, 3), (N, N, 3)).reshape(NN, 3)
        diff = pi - pj
        d = jnp.sqrt(jnp.sum(diff * diff, axis=1, keepdims=True) + 1e-12)
        maskf = jnp.where((d < _CUT) & (~same), 1.0, 0.0).astype(f32)
        cc = _cos_cut(d) * maskf
        b_list.append(_catmull_basis(d * np.float32(1.0 / _DELTA), riota) * cc)
    bcat = jnp.concatenate(b_list, axis=1).astype(bf16)

    zc = zc_ref[0]
    ioz = jax.lax.broadcasted_iota(jnp.int32, (N, _MAXZ), 1)
    ohc = jnp.concatenate(
        [(zc[0] == ioz).astype(f32), (zc[1] == ioz).astype(f32)], axis=1)
    h = jax.lax.dot_general(ohc, emb_ref[:, :], (((1,), (0,)), ((), ())),
                            preferred_element_type=f32, precision=_HI)

    ws = [jnp.dot(bcat, td_ref[i], preferred_element_type=f32)
          for i in range(_NI)]
    for i in range(_NI):
        xj = jnp.dot(h, l1_ref[i], preferred_element_type=f32)
        w = ws[i]
        agg = jnp.sum(w.reshape(N, N, 2 * _FILT) * xj[None, :, :], axis=1)
        m = _ssp(jnp.dot(agg, l2_ref[i], preferred_element_type=f32) + bl2_ref[i])
        m = jnp.dot(m, l_ref[i], preferred_element_type=f32) + bl_ref[i]
        h = h + m

    o = _ssp(jnp.dot(h, o1_ref[:, :], preferred_element_type=f32) + bo1_ref[:, :])
    s = jnp.sum(o, axis=0, keepdims=True)
    e = (jnp.dot(s, o2_ref[:, :], preferred_element_type=f32, precision=_HI)
         + float(N) * bo2_ref[:, :])
    y = jnp.maximum(
        jnp.dot(e, h1w_ref[:, :], preferred_element_type=f32, precision=_HI)
        + h1b_ref[:, :], 0.0)
    y = (jnp.dot(y, h2w_ref[:, :], preferred_element_type=f32, precision=_HI)
         + h2b_ref[:, :])
    out_ref[:, :, :] = y.reshape(1, 1, 1)


def _bdiag(a, b):
    ka, na = a.shape
    kb, nb = b.shape
    return jnp.concatenate(
        [jnp.concatenate([a, jnp.zeros((ka, nb), jnp.float32)], 1),
         jnp.concatenate([jnp.zeros((kb, na), jnp.float32), b], 1)], 0)


def kernel(z, pos, params1, params2, head):
    B = z.shape[0]
    zq = z.reshape(B, 2, _N, 1).astype(jnp.int32)
    pq = pos.reshape(B, 2, _N, 3).astype(jnp.float32)

    i1 = params1["inter"]
    i2 = params2["inter"]
    W1s = jnp.stack([_bdiag(i1[i]["mlp1"]["w"], i2[i]["mlp1"]["w"]) for i in range(_NI)])
    B1s = jnp.stack([jnp.concatenate([i1[i]["mlp1"]["b"], i2[i]["mlp1"]["b"]])[None, :] for i in range(_NI)])
    W2s = jnp.stack([_bdiag(i1[i]["mlp2"]["w"], i2[i]["mlp2"]["w"]) for i in range(_NI)])
    B2s = jnp.stack([jnp.concatenate([i1[i]["mlp2"]["b"], i2[i]["mlp2"]["b"]])[None, :] for i in range(_NI)])
    L1s = jnp.stack([_bdiag(i1[i]["lin1"]["w"], i2[i]["lin1"]["w"]) for i in range(_NI)])
    L2s = jnp.stack([_bdiag(i1[i]["lin2"]["w"], i2[i]["lin2"]["w"]) for i in range(_NI)])
    BL2s = jnp.stack([jnp.concatenate([i1[i]["lin2"]["b"], i2[i]["lin2"]["b"]])[None, :] for i in range(_NI)])
    Ls = jnp.stack([_bdiag(i1[i]["lin"]["w"], i2[i]["lin"]["w"]) for i in range(_NI)])
    BLs = jnp.stack([jnp.concatenate([i1[i]["lin"]["b"], i2[i]["lin"]["b"]])[None, :] for i in range(_NI)])
    EMB = _bdiag(params1["embed"], params2["embed"])
    O1 = _bdiag(params1["out1"]["w"], params2["out1"]["w"])
    BO1 = jnp.concatenate([params1["out1"]["b"], params2["out1"]["b"]])[None, :]
    O2 = _bdiag(params1["out2"]["w"], params2["out2"]["w"])
    BO2 = jnp.concatenate([params1["out2"]["b"], params2["out2"]["b"]])[None, :]
    H1W = head["l1"]["w"]
    H1B = head["l1"]["b"][None, :]
    H2W = head["l2"]["w"]
    H2B = head["l2"]["b"][None, :]

    def full(a):
        return pl.BlockSpec(a.shape, lambda b, nd=a.ndim: (0,) * nd)

    tconsts = (W1s, B1s, W2s, B2s)

    def full0(a):
        return pl.BlockSpec(a.shape, lambda nd=a.ndim: (0,) * nd)

    TD = pl.pallas_call(
        _table_kernel,
        in_specs=[full0(a) for a in tconsts],
        out_specs=pl.BlockSpec((_NI, 2 * _TROWS, 2 * _FILT), lambda: (0, 0, 0)),
        out_shape=jax.ShapeDtypeStruct((_NI, 2 * _TROWS, 2 * _FILT),
                                       jnp.bfloat16),
    )(*tconsts)

    consts = (EMB, TD, L1s, L2s, BL2s, Ls, BLs,
              O1, BO1, O2, BO2, H1W, H1B, H2W, H2B)
    out = pl.pallas_call(
        _pair_kernel,
        grid=(B,),
        in_specs=[
            pl.BlockSpec((1, 2, _N, 1), lambda b: (b, 0, 0, 0)),
            pl.BlockSpec((1, 2, _N, 3), lambda b: (b, 0, 0, 0)),
        ] + [full(a) for a in consts],
        out_specs=pl.BlockSpec((1, 1, 1), lambda b: (b, 0, 0)),
        out_shape=jax.ShapeDtypeStruct((B, 1, 1), jnp.float32),
        compiler_params=pltpu.CompilerParams(dimension_semantics=("parallel",)),
    )(zq, pq, *consts)
    return out.reshape(B, 1)


# speedup vs baseline: 3.4494x; 1.1108x over previous
import numpy as np
import jax
import jax.numpy as jnp
from jax.experimental import pallas as pl
from jax.experimental.pallas import tpu as pltpu

_HIDDEN = 128
_FILT = 128
_NG = 50
_NI = 6
_CUT = 10.0
_MAXZ = 100
_N = 64
_LN2 = 0.6931471805599453

_OFFS = np.linspace(0.0, _CUT, _NG).astype(np.float32)
_COEFF = float(-0.5 / (_OFFS[1] - _OFFS[0]) ** 2)

_TROWS = 128
_DELTA = float(_CUT / (_TROWS - 3))

_HI = jax.lax.Precision.HIGHEST

_COS_COEF = (0.9999997217202586, -2.4673792800171435, 2.0290814841625955,
             -0.6663741230530931, 0.11506268303889056, -0.010391673928505263)


def _cos_cut_sq(s):
    p = jnp.float32(_COS_COEF[5])
    for k in (4, 3, 2, 1, 0):
        p = p * s + _COS_COEF[k]
    return p


def _ssp(x):
    return jnp.maximum(x, 0.0) + jnp.log1p(jnp.exp(-jnp.abs(x))) - _LN2


def _table_kernel(w1_ref, b1_ref, w2_ref, b2_ref, t_ref):
    f32 = jnp.float32
    offs = (jax.lax.broadcasted_iota(jnp.int32, (1, _NG), 1).astype(f32)
            * np.float32(_CUT / (_NG - 1)))
    dk = (jax.lax.broadcasted_iota(jnp.int32, (_TROWS, 1), 0).astype(f32)
          - 1.0) * np.float32(_DELTA)
    rb = jnp.exp(_COEFF * (dk - offs) ** 2)
    rbc = jnp.concatenate([rb, rb], axis=1)
    zpad = jnp.zeros((_TROWS, _FILT), jnp.bfloat16)
    for i in range(_NI):
        t = _ssp(jnp.dot(rbc, w1_ref[i], preferred_element_type=f32,
                         precision=_HI) + b1_ref[i])
        t = (jnp.dot(t, w2_ref[i], preferred_element_type=f32,
                     precision=_HI) + b2_ref[i]).astype(jnp.bfloat16)
        t_ref[i] = jnp.concatenate(
            [jnp.concatenate([t[:, :_FILT], zpad], axis=1),
             jnp.concatenate([zpad, t[:, _FILT:]], axis=1)], axis=0)


def _catmull_basis(t, riota):
    x = t - riota
    a = jnp.abs(x)
    a2 = a * a
    inner = (1.5 * a - 2.5) * a2 + 1.0
    outer = ((-0.5 * a + 2.5) * a - 4.0) * a + 2.0
    w = jnp.where(a < 1.0, inner, outer)
    return jnp.where(a < 2.0, w, 0.0)


def _pair_kernel(zc_ref, pos_ref, emb_ref, td_ref,
                 l1_ref, l2_ref, bl2_ref, l_ref, bl_ref,
                 o1_ref, bo1_ref, o2_ref, bo2_ref,
                 h1w_ref, h1b_ref, h2w_ref, h2b_ref, out_ref):
    f32 = jnp.float32
    bf16 = jnp.bfloat16
    N = _N
    NN = N * N
    pos = pos_ref[0]

    same = (jax.lax.broadcasted_iota(jnp.int32, (N, N, 1), 0)
            == jax.lax.broadcasted_iota(jnp.int32, (N, N, 1), 1)).reshape(NN, 1)
    riota = (jax.lax.broadcasted_iota(jnp.int32, (1, _TROWS), 1).astype(f32)
             - 1.0)

    d2_list = []
    for c in range(2):
        p = pos[c]
        pi = jnp.broadcast_to(p.reshape(N, 1, 3), (N, N, 3)).reshape(NN, 3)
        pj = jnp.broadcast_to(p.reshape(1, N, 3), (N, N, 3)).reshape(NN, 3)
        diff = pi - pj
        d2_list.append(jnp.sum(diff * diff, axis=1, keepdims=True))
    d2b = jnp.concatenate(d2_list, axis=1)
    maskb = jnp.where((d2b < _CUT * _CUT) & (~same), 1.0, 0.0).astype(f32)
    ccb = _cos_cut_sq((d2b + 1e-12) * np.float32(1.0 / (_CUT * _CUT))) * maskb
    tb = jnp.sqrt(d2b + 1e-12) * np.float32(1.0 / _DELTA)
    bcat = jnp.concatenate(
        [_catmull_basis(tb[:, 0:1], riota) * ccb[:, 0:1],
         _catmull_basis(tb[:, 1:2], riota) * ccb[:, 1:2]],
        axis=1).astype(bf16)

    zc = zc_ref[0]
    ioz = jax.lax.broadcasted_iota(jnp.int32, (N, _MAXZ), 1)
    ohc = jnp.concatenate(
        [(zc[0] == ioz).astype(f32), (zc[1] == ioz).astype(f32)], axis=1)
    h = jax.lax.dot_general(ohc, emb_ref[:, :], (((1,), (0,)), ((), ())),
                            preferred_element_type=f32, precision=_HI)

    ws = [jnp.dot(bcat, td_ref[i], preferred_element_type=f32)
          for i in range(_NI)]
    for i in range(_NI):
        xj = jnp.dot(h, l1_ref[i], preferred_element_type=f32)
        w = ws[i]
        agg = jnp.sum(w.reshape(N, N, 2 * _FILT) * xj[None, :, :], axis=1)
        m = _ssp(jnp.dot(agg, l2_ref[i], preferred_element_type=f32) + bl2_ref[i])
        m = jnp.dot(m, l_ref[i], preferred_element_type=f32) + bl_ref[i]
        h = h + m

    o = _ssp(jnp.dot(h, o1_ref[:, :], preferred_element_type=f32) + bo1_ref[:, :])
    s = jnp.sum(o, axis=0, keepdims=True)
    e = (jnp.dot(s, o2_ref[:, :], preferred_element_type=f32, precision=_HI)
         + float(N) * bo2_ref[:, :])
    y = jnp.maximum(
        jnp.dot(e, h1w_ref[:, :], preferred_element_type=f32, precision=_HI)
        + h1b_ref[:, :], 0.0)
    y = (jnp.dot(y, h2w_ref[:, :], preferred_element_type=f32, precision=_HI)
         + h2b_ref[:, :])
    out_ref[:, :, :] = y.reshape(1, 1, 1)


def _bdiag(a, b):
    ka, na = a.shape
    kb, nb = b.shape
    return jnp.concatenate(
        [jnp.concatenate([a, jnp.zeros((ka, nb), jnp.float32)], 1),
         jnp.concatenate([jnp.zeros((kb, na), jnp.float32), b], 1)], 0)


def kernel(z, pos, params1, params2, head):
    B = z.shape[0]
    zq = z.reshape(B, 2, _N, 1).astype(jnp.int32)
    pq = pos.reshape(B, 2, _N, 3).astype(jnp.float32)

    i1 = params1["inter"]
    i2 = params2["inter"]
    W1s = jnp.stack([_bdiag(i1[i]["mlp1"]["w"], i2[i]["mlp1"]["w"]) for i in range(_NI)])
    B1s = jnp.stack([jnp.concatenate([i1[i]["mlp1"]["b"], i2[i]["mlp1"]["b"]])[None, :] for i in range(_NI)])
    W2s = jnp.stack([_bdiag(i1[i]["mlp2"]["w"], i2[i]["mlp2"]["w"]) for i in range(_NI)])
    B2s = jnp.stack([jnp.concatenate([i1[i]["mlp2"]["b"], i2[i]["mlp2"]["b"]])[None, :] for i in range(_NI)])
    L1s = jnp.stack([_bdiag(i1[i]["lin1"]["w"], i2[i]["lin1"]["w"]) for i in range(_NI)])
    L2s = jnp.stack([_bdiag(i1[i]["lin2"]["w"], i2[i]["lin2"]["w"]) for i in range(_NI)])
    BL2s = jnp.stack([jnp.concatenate([i1[i]["lin2"]["b"], i2[i]["lin2"]["b"]])[None, :] for i in range(_NI)])
    Ls = jnp.stack([_bdiag(i1[i]["lin"]["w"], i2[i]["lin"]["w"]) for i in range(_NI)])
    BLs = jnp.stack([jnp.concatenate([i1[i]["lin"]["b"], i2[i]["lin"]["b"]])[None, :] for i in range(_NI)])
    EMB = _bdiag(params1["embed"], params2["embed"])
    O1 = _bdiag(params1["out1"]["w"], params2["out1"]["w"])
    BO1 = jnp.concatenate([params1["out1"]["b"], params2["out1"]["b"]])[None, :]
    O2 = _bdiag(params1["out2"]["w"], params2["out2"]["w"])
    BO2 = jnp.concatenate([params1["out2"]["b"], params2["out2"]["b"]])[None, :]
    H1W = head["l1"]["w"]
    H1B = head["l1"]["b"][None, :]
    H2W = head["l2"]["w"]
    H2B = head["l2"]["b"][None, :]

    def full(a):
        return pl.BlockSpec(a.shape, lambda b, nd=a.ndim: (0,) * nd)

    tconsts = (W1s, B1s, W2s, B2s)

    def full0(a):
        return pl.BlockSpec(a.shape, lambda nd=a.ndim: (0,) * nd)

    TD = pl.pallas_call(
        _table_kernel,
        in_specs=[full0(a) for a in tconsts],
        out_specs=pl.BlockSpec((_NI, 2 * _TROWS, 2 * _FILT), lambda: (0, 0, 0)),
        out_shape=jax.ShapeDtypeStruct((_NI, 2 * _TROWS, 2 * _FILT),
                                       jnp.bfloat16),
    )(*tconsts)

    consts = (EMB, TD, L1s, L2s, BL2s, Ls, BLs,
              O1, BO1, O2, BO2, H1W, H1B, H2W, H2B)
    out = pl.pallas_call(
        _pair_kernel,
        grid=(B,),
        in_specs=[
            pl.BlockSpec((1, 2, _N, 1), lambda b: (b, 0, 0, 0)),
            pl.BlockSpec((1, 2, _N, 3), lambda b: (b, 0, 0, 0)),
        ] + [full(a) for a in consts],
        out_specs=pl.BlockSpec((1, 1, 1), lambda b: (b, 0, 0)),
        out_shape=jax.ShapeDtypeStruct((B, 1, 1), jnp.float32),
        compiler_params=pltpu.CompilerParams(dimension_semantics=("parallel",)),
    )(zq, pq, *consts)
    return out.reshape(B, 1)


# speedup vs baseline: 3.9805x; 1.1540x over previous
import numpy as np
import jax
import jax.numpy as jnp
from jax.experimental import pallas as pl
from jax.experimental.pallas import tpu as pltpu

_HIDDEN = 128
_FILT = 128
_NG = 50
_NI = 6
_CUT = 10.0
_MAXZ = 100
_N = 64
_LN2 = 0.6931471805599453

_OFFS = np.linspace(0.0, _CUT, _NG).astype(np.float32)
_COEFF = float(-0.5 / (_OFFS[1] - _OFFS[0]) ** 2)

_TROWS = 128
_DELTA = float(_CUT / (_TROWS - 3))

_HI = jax.lax.Precision.HIGHEST

_COS_COEF = (0.9999997217202586, -2.4673792800171435, 2.0290814841625955,
             -0.6663741230530931, 0.11506268303889056, -0.010391673928505263)


def _cos_cut_sq(s):
    p = jnp.float32(_COS_COEF[5])
    for k in (4, 3, 2, 1, 0):
        p = p * s + _COS_COEF[k]
    return p


def _ssp(x):
    return jnp.maximum(x, 0.0) + jnp.log1p(jnp.exp(-jnp.abs(x))) - _LN2


def _table_kernel(w1_ref, b1_ref, w2_ref, b2_ref, t_ref):
    f32 = jnp.float32
    offs = (jax.lax.broadcasted_iota(jnp.int32, (1, _NG), 1).astype(f32)
            * np.float32(_CUT / (_NG - 1)))
    dk = (jax.lax.broadcasted_iota(jnp.int32, (_TROWS, 1), 0).astype(f32)
          - 1.0) * np.float32(_DELTA)
    rb = jnp.exp(_COEFF * (dk - offs) ** 2)
    rbc = jnp.concatenate([rb, rb], axis=1)
    zpad = jnp.zeros((_TROWS, _FILT), jnp.bfloat16)
    for i in range(_NI):
        t = _ssp(jnp.dot(rbc, w1_ref[i], preferred_element_type=f32,
                         precision=_HI) + b1_ref[i])
        t = (jnp.dot(t, w2_ref[i], preferred_element_type=f32,
                     precision=_HI) + b2_ref[i]).astype(jnp.bfloat16)
        t_ref[i] = jnp.concatenate(
            [jnp.concatenate([t[:, :_FILT], zpad], axis=1),
             jnp.concatenate([zpad, t[:, _FILT:]], axis=1)], axis=0)


def _catmull_basis(t, riota):
    x = t - riota
    a = jnp.abs(x)
    a2 = a * a
    inner = (1.5 * a - 2.5) * a2 + 1.0
    outer = ((-0.5 * a + 2.5) * a - 4.0) * a + 2.0
    w = jnp.where(a < 1.0, inner, outer)
    return jnp.where(a < 2.0, w, 0.0)


def _pair_kernel(zc_ref, pos_ref, emb_ref, td_ref,
                 l1_ref, l2_ref, bl2_ref, l_ref, bl_ref,
                 o1_ref, bo1_ref, o2_ref, bo2_ref,
                 h1w_ref, h1b_ref, h2w_ref, h2b_ref, out_ref):
    f32 = jnp.float32
    bf16 = jnp.bfloat16
    N = _N
    NN = N * N
    pos = pos_ref[0]

    same = (jax.lax.broadcasted_iota(jnp.int32, (N, N, 1), 0)
            == jax.lax.broadcasted_iota(jnp.int32, (N, N, 1), 1)).reshape(NN, 1)
    riota = (jax.lax.broadcasted_iota(jnp.int32, (1, _TROWS), 1).astype(f32)
             - 1.0)

    d2_list = []
    for c in range(2):
        p = pos[c]
        pi = jnp.broadcast_to(p.reshape(N, 1, 3), (N, N, 3)).reshape(NN, 3)
        pj = jnp.broadcast_to(p.reshape(1, N, 3), (N, N, 3)).reshape(NN, 3)
        diff = pi - pj
        d2_list.append(jnp.sum(diff * diff, axis=1, keepdims=True))
    d2b = jnp.concatenate(d2_list, axis=1)
    maskb = jnp.where((d2b < _CUT * _CUT) & (~same), 1.0, 0.0).astype(f32)
    ccb = _cos_cut_sq((d2b + 1e-12) * np.float32(1.0 / (_CUT * _CUT))) * maskb
    tb = jnp.sqrt(d2b + 1e-12) * np.float32(1.0 / _DELTA)
    bcat = jnp.concatenate(
        [_catmull_basis(tb[:, 0:1], riota) * ccb[:, 0:1],
         _catmull_basis(tb[:, 1:2], riota) * ccb[:, 1:2]],
        axis=1).astype(bf16)

    zc = zc_ref[0]
    ioz = jax.lax.broadcasted_iota(jnp.int32, (N, _MAXZ), 1)
    ohc = jnp.concatenate(
        [(zc[0] == ioz).astype(f32), (zc[1] == ioz).astype(f32)], axis=1)
    h = jax.lax.dot_general(ohc, emb_ref[:, :], (((1,), (0,)), ((), ())),
                            preferred_element_type=f32, precision=_HI)

    ws = [jnp.dot(bcat, td_ref[i], preferred_element_type=f32)
          for i in range(_NI)]
    for i in range(_NI):
        xj = jnp.dot(h, l1_ref[i], preferred_element_type=f32)
        w = ws[i]
        agg = jnp.sum(w.reshape(N, N, 2 * _FILT) * xj[:, None, :], axis=0)
        m = _ssp(jnp.dot(agg, l2_ref[i], preferred_element_type=f32) + bl2_ref[i])
        m = jnp.dot(m, l_ref[i], preferred_element_type=f32) + bl_ref[i]
        h = h + m

    o = _ssp(jnp.dot(h, o1_ref[:, :], preferred_element_type=f32) + bo1_ref[:, :])
    s = jnp.sum(o, axis=0, keepdims=True)
    e = (jnp.dot(s, o2_ref[:, :], preferred_element_type=f32, precision=_HI)
         + float(N) * bo2_ref[:, :])
    y = jnp.maximum(
        jnp.dot(e, h1w_ref[:, :], preferred_element_type=f32, precision=_HI)
        + h1b_ref[:, :], 0.0)
    y = (jnp.dot(y, h2w_ref[:, :], preferred_element_type=f32, precision=_HI)
         + h2b_ref[:, :])
    out_ref[:, :, :] = y.reshape(1, 1, 1)


def _bdiag(a, b):
    ka, na = a.shape
    kb, nb = b.shape
    return jnp.concatenate(
        [jnp.concatenate([a, jnp.zeros((ka, nb), jnp.float32)], 1),
         jnp.concatenate([jnp.zeros((kb, na), jnp.float32), b], 1)], 0)


def kernel(z, pos, params1, params2, head):
    B = z.shape[0]
    zq = z.reshape(B, 2, _N, 1).astype(jnp.int32)
    pq = pos.reshape(B, 2, _N, 3).astype(jnp.float32)

    i1 = params1["inter"]
    i2 = params2["inter"]
    W1s = jnp.stack([_bdiag(i1[i]["mlp1"]["w"], i2[i]["mlp1"]["w"]) for i in range(_NI)])
    B1s = jnp.stack([jnp.concatenate([i1[i]["mlp1"]["b"], i2[i]["mlp1"]["b"]])[None, :] for i in range(_NI)])
    W2s = jnp.stack([_bdiag(i1[i]["mlp2"]["w"], i2[i]["mlp2"]["w"]) for i in range(_NI)])
    B2s = jnp.stack([jnp.concatenate([i1[i]["mlp2"]["b"], i2[i]["mlp2"]["b"]])[None, :] for i in range(_NI)])
    L1s = jnp.stack([_bdiag(i1[i]["lin1"]["w"], i2[i]["lin1"]["w"]) for i in range(_NI)])
    L2s = jnp.stack([_bdiag(i1[i]["lin2"]["w"], i2[i]["lin2"]["w"]) for i in range(_NI)])
    BL2s = jnp.stack([jnp.concatenate([i1[i]["lin2"]["b"], i2[i]["lin2"]["b"]])[None, :] for i in range(_NI)])
    Ls = jnp.stack([_bdiag(i1[i]["lin"]["w"], i2[i]["lin"]["w"]) for i in range(_NI)])
    BLs = jnp.stack([jnp.concatenate([i1[i]["lin"]["b"], i2[i]["lin"]["b"]])[None, :] for i in range(_NI)])
    EMB = _bdiag(params1["embed"], params2["embed"])
    O1 = _bdiag(params1["out1"]["w"], params2["out1"]["w"])
    BO1 = jnp.concatenate([params1["out1"]["b"], params2["out1"]["b"]])[None, :]
    O2 = _bdiag(params1["out2"]["w"], params2["out2"]["w"])
    BO2 = jnp.concatenate([params1["out2"]["b"], params2["out2"]["b"]])[None, :]
    H1W = head["l1"]["w"]
    H1B = head["l1"]["b"][None, :]
    H2W = head["l2"]["w"]
    H2B = head["l2"]["b"][None, :]

    def full(a):
        return pl.BlockSpec(a.shape, lambda b, nd=a.ndim: (0,) * nd)

    tconsts = (W1s, B1s, W2s, B2s)

    def full0(a):
        return pl.BlockSpec(a.shape, lambda nd=a.ndim: (0,) * nd)

    TD = pl.pallas_call(
        _table_kernel,
        in_specs=[full0(a) for a in tconsts],
        out_specs=pl.BlockSpec((_NI, 2 * _TROWS, 2 * _FILT), lambda: (0, 0, 0)),
        out_shape=jax.ShapeDtypeStruct((_NI, 2 * _TROWS, 2 * _FILT),
                                       jnp.bfloat16),
    )(*tconsts)

    consts = (EMB, TD, L1s, L2s, BL2s, Ls, BLs,
              O1, BO1, O2, BO2, H1W, H1B, H2W, H2B)
    out = pl.pallas_call(
        _pair_kernel,
        grid=(B,),
        in_specs=[
            pl.BlockSpec((1, 2, _N, 1), lambda b: (b, 0, 0, 0)),
            pl.BlockSpec((1, 2, _N, 3), lambda b: (b, 0, 0, 0)),
        ] + [full(a) for a in consts],
        out_specs=pl.BlockSpec((1, 1, 1), lambda b: (b, 0, 0)),
        out_shape=jax.ShapeDtypeStruct((B, 1, 1), jnp.float32),
        compiler_params=pltpu.CompilerParams(dimension_semantics=("arbitrary",)),
    )(zq, pq, *consts)
    return out.reshape(B, 1)
